# baseline (speedup 1.0000x reference)
import functools

import jax
import jax.numpy as jnp
from jax import lax
from jax.experimental import pallas as pl
from jax.experimental.pallas import tpu as pltpu
from jax.experimental.pallas import tpu_sc as plsc

N = 10000
E = 320000
D = 128
G = 64
NC, NS, L = 2, 16, 16

CH = 128
NCHUNK = 79
E_PAD = 32 * NCHUNK * CH
SCHUNK = 2 * NCHUNK
K0 = 128
K1 = SCHUNK - K0
KMAX = max(K0, K1)
SLAB = K0 + KMAX
NROWS = 10112
ZROWS = 632
BN = 1000
NBLK = N // BN


@functools.cache
def _sc_scatter_kernel():
    mesh = plsc.VectorSubcoreMesh(core_axis_name="c", subcore_axis_name="s")
    return functools.partial(
        pl.kernel,
        out_type=jax.ShapeDtypeStruct((NC, NROWS, D), jnp.float32),
        mesh=mesh,
        scratch_types=[
            pltpu.VMEM_SHARED((NROWS, D), jnp.float32),
            pltpu.VMEM((KMAX, CH), jnp.int32),
            pltpu.VMEM((KMAX, CH), jnp.int32),
            pltpu.VMEM((CH, D), jnp.float32),
            pltpu.SemaphoreType.DMA,
        ],
    )(_sc_scatter_body)


def _sc_scatter_body(table, srcC, dstC, zer, acc3,
                     acc_sp, srcv, dstv, gbuf, sem):
    c = lax.axis_index("c")
    s = lax.axis_index("s")
    start = jnp.where(c == 0, 0, K0)
    count = jnp.where(c == 0, K0, K1)
    pltpu.sync_copy(srcC.at[s, pl.ds(start, KMAX)], srcv)
    pltpu.sync_copy(dstC.at[s, pl.ds(start, KMAX)], dstv)
    pltpu.sync_copy(zer, acc_sp.at[pl.ds(s * ZROWS, ZROWS)])
    plsc.subcore_barrier()

    def body(j, _):
        pltpu.async_copy(table.at[srcv.at[j]], gbuf, sem).wait()
        pltpu.sync_copy(gbuf, acc_sp.at[dstv.at[j]], add=True)
        return 0

    lax.fori_loop(0, count, body, 0)
    plsc.subcore_barrier()
    pltpu.sync_copy(acc_sp.at[pl.ds(s * ZROWS, ZROWS)],
                    acc3.at[c, pl.ds(s * ZROWS, ZROWS)])


EPT = E_PAD // 32
DROWS = NROWS // NS


@functools.cache
def _sc_degree_kernel():
    mesh = plsc.VectorSubcoreMesh(core_axis_name="c", subcore_axis_name="s")
    return functools.partial(
        pl.kernel,
        out_type=jax.ShapeDtypeStruct((NC, NROWS, L), jnp.float32),
        mesh=mesh,
        compiler_params=pltpu.CompilerParams(needs_layout_passes=False),
        scratch_types=[
            pltpu.VMEM_SHARED((NS * NROWS,), jnp.float32),
            pltpu.VMEM((NROWS,), jnp.float32),
            pltpu.VMEM((DROWS + 8,), jnp.float32),
            pltpu.VMEM((DROWS + 8,), jnp.float32),
            pltpu.VMEM((EPT,), jnp.int32),
            pltpu.VMEM((DROWS, L), jnp.float32),
        ],
    )(_sc_degree_body)


def _sc_degree_body(dstF, deg3, tabs_sp, hist_v, sum_v, tmp_v, dstv, rep_v):
    c = lax.axis_index("c")
    s = lax.axis_index("s")
    wid = s * NC + c
    pltpu.sync_copy(dstF.at[wid], dstv)
    zeros16 = jnp.zeros((L,), jnp.float32)
    ones16 = jnp.ones((L,), jnp.float32)

    def zbody(j, _):
        hist_v[pl.ds(j * L, L)] = zeros16
        return 0

    lax.fori_loop(0, NROWS // L, zbody, 0)

    def hbody(j, _):
        idx = dstv[pl.ds(j * L, L)]
        plsc.addupdate_scatter(hist_v, [idx], ones16)
        return 0

    lax.fori_loop(0, EPT // L, hbody, 0)

    pltpu.sync_copy(hist_v, tabs_sp.at[pl.ds(s * NROWS, NROWS)])
    plsc.subcore_barrier()
    pltpu.sync_copy(tabs_sp.at[pl.ds(s * DROWS, DROWS)],
                    sum_v.at[pl.ds(0, DROWS)])
    for t in range(1, NS):
        pltpu.sync_copy(tabs_sp.at[pl.ds(t * NROWS + s * DROWS, DROWS)],
                        tmp_v.at[pl.ds(0, DROWS)])

        def abody(k, _):
            sl = pl.ds(k * L, L)
            sum_v[sl] = sum_v[sl] + tmp_v[sl]
            return 0

        lax.fori_loop(0, (DROWS + 8) // L, abody, 0)

    def rbody(j, _):
        v = plsc.load_gather(sum_v, [jnp.full((L,), j, jnp.int32)])
        plsc.store_scatter(
            rep_v, [jnp.full((L,), j, jnp.int32), lax.iota(jnp.int32, L)], v)
        return 0

    lax.fori_loop(0, DROWS, rbody, 0)
    pltpu.sync_copy(rep_v, deg3.at[c, pl.ds(s * DROWS, DROWS)])


def _dinv_block(degA, degB):
    deg = degA[0][:, :1] + degB[0][:, :1] + 1.0
    return lax.rsqrt(deg)


def _deg_specs():
    return [
        pl.BlockSpec((1, BN, L), lambda i: (0, i, 0)),
        pl.BlockSpec((1, BN, L), lambda i: (1, i, 0)),
    ]


def _tc1_body(x_ref, w1_ref, degA_ref, degB_ref, hs_ref):
    dinv = _dinv_block(degA_ref[...], degB_ref[...])
    h = jnp.dot(x_ref[...], w1_ref[...], preferred_element_type=jnp.float32)
    hs_ref[...] = h * dinv


def _tc1(x, W1, deg3):
    return pl.pallas_call(
        _tc1_body,
        grid=(NBLK,),
        in_specs=[
            pl.BlockSpec((BN, D), lambda i: (i, 0)),
            pl.BlockSpec((D, D), lambda i: (0, 0)),
        ] + _deg_specs(),
        out_specs=pl.BlockSpec((BN, D), lambda i: (i, 0)),
        out_shape=jax.ShapeDtypeStruct((N, D), jnp.float32),
    )(x, W1, deg3, deg3)


def _tc2_body(accA_ref, accB_ref, hs_ref, degA_ref, degB_ref,
              b1_ref, w2_ref, o_ref):
    dinv = _dinv_block(degA_ref[...], degB_ref[...])
    acc = accA_ref[0] + accB_ref[0]
    h1 = jax.nn.relu(dinv * (acc + hs_ref[...]) + b1_ref[...])
    o_ref[...] = jnp.dot(h1, w2_ref[...],
                         preferred_element_type=jnp.float32) * dinv


def _tc2(acc3, hs, deg3, b1, W2):
    return pl.pallas_call(
        _tc2_body,
        grid=(NBLK,),
        in_specs=[
            pl.BlockSpec((1, BN, D), lambda i: (0, i, 0)),
            pl.BlockSpec((1, BN, D), lambda i: (1, i, 0)),
            pl.BlockSpec((BN, D), lambda i: (i, 0)),
        ] + _deg_specs() + [
            pl.BlockSpec((1, D), lambda i: (0, 0)),
            pl.BlockSpec((D, D), lambda i: (0, 0)),
        ],
        out_specs=pl.BlockSpec((BN, D), lambda i: (i, 0)),
        out_shape=jax.ShapeDtypeStruct((N, D), jnp.float32),
    )(acc3, acc3, hs, deg3, deg3, b1, W2)


def _tc3_body(accA_ref, accB_ref, hs_ref, degA_ref, degB_ref,
              b2_ref, batch_ref, wf1_ref, bf1_ref, wf2_ref, bf2_ref,
              out_ref, pool_s, cnt_s):
    i = pl.program_id(0)
    dinv = _dinv_block(degA_ref[...], degB_ref[...])
    acc = accA_ref[0] + accB_ref[0]
    h2 = jax.nn.relu(dinv * (acc + hs_ref[...]) + b2_ref[...])

    @pl.when(i == 0)
    def _():
        pool_s[...] = jnp.zeros_like(pool_s)
        cnt_s[...] = jnp.zeros_like(cnt_s)

    ids = batch_ref[0, 0, :]
    gids = lax.broadcasted_iota(jnp.int32, (G, BN), 0)
    onehot = (ids[None, :] == gids).astype(jnp.float32)
    pool_s[...] += jnp.dot(onehot, h2, preferred_element_type=jnp.float32)
    cnt_s[...] += jnp.dot(onehot, jnp.ones((BN, D), jnp.float32),
                          preferred_element_type=jnp.float32)

    @pl.when(i == NBLK - 1)
    def _():
        pooled = pool_s[...] / jnp.maximum(cnt_s[...], 1.0)
        o = jax.nn.relu(
            jnp.dot(pooled, wf1_ref[...], preferred_element_type=jnp.float32)
            + bf1_ref[...])
        out_ref[...] = (
            jnp.dot(o, wf2_ref[...], preferred_element_type=jnp.float32)
            + bf2_ref[...])


def _tc3(acc3, hs, deg3, b2, batch3, Wf1, bf1, Wf2p, bf2p):
    return pl.pallas_call(
        _tc3_body,
        grid=(NBLK,),
        in_specs=[
            pl.BlockSpec((1, BN, D), lambda i: (0, i, 0)),
            pl.BlockSpec((1, BN, D), lambda i: (1, i, 0)),
            pl.BlockSpec((BN, D), lambda i: (i, 0)),
        ] + _deg_specs() + [
            pl.BlockSpec((1, D), lambda i: (0, 0)),
            pl.BlockSpec((1, 1, BN), lambda i: (i, 0, 0)),
            pl.BlockSpec((D, G), lambda i: (0, 0)),
            pl.BlockSpec((1, G), lambda i: (0, 0)),
            pl.BlockSpec((G, D), lambda i: (0, 0)),
            pl.BlockSpec((1, D), lambda i: (0, 0)),
        ],
        out_specs=pl.BlockSpec((G, D), lambda i: (0, 0)),
        out_shape=jax.ShapeDtypeStruct((G, D), jnp.float32),
        scratch_shapes=[
            pltpu.VMEM((G, D), jnp.float32),
            pltpu.VMEM((G, D), jnp.float32),
        ],
    )(acc3, acc3, hs, deg3, deg3, b2, batch3, Wf1, bf1, Wf2p, bf2p)


def kernel(x, edge_index, batch, W1, b1, W2, b2, Wf1, bf1, Wf2, bf2):
    src = edge_index[0].astype(jnp.int32)
    dst = edge_index[1].astype(jnp.int32)
    npad = E_PAD - E
    src_pad = jnp.concatenate([src, jnp.zeros((npad,), jnp.int32)])
    dst_pad = jnp.concatenate([dst, jnp.full((npad,), N, jnp.int32)])

    src2 = src_pad.reshape(NS, SCHUNK, CH)
    dst2 = dst_pad.reshape(NS, SCHUNK, CH)
    padc = jnp.zeros((NS, SLAB - SCHUNK, CH), jnp.int32)
    srcC = jnp.concatenate([src2, padc], axis=1)
    dstC = jnp.concatenate([dst2, jnp.full((NS, SLAB - SCHUNK, CH), N,
                                           jnp.int32)], axis=1)

    dstF = dst_pad.reshape(32, EPT)
    zer128 = jnp.zeros((ZROWS, D), jnp.float32)
    batch3 = batch.astype(jnp.int32).reshape(NBLK, 1, BN)

    scat = _sc_scatter_kernel()
    deg3 = _sc_degree_kernel()(dstF)
    hs = _tc1(x, W1, deg3)
    acc3 = scat(hs, srcC, dstC, zer128)
    hs2 = _tc2(acc3, hs, deg3, b1.reshape(1, D), W2)
    acc23 = scat(hs2, srcC, dstC, zer128)
    Wf2p = jnp.zeros((G, D), jnp.float32).at[:, :2].set(Wf2)
    bf2p = jnp.zeros((1, D), jnp.float32).at[:, :2].set(bf2.reshape(1, 2))
    outp = _tc3(acc23, hs2, deg3, b2.reshape(1, D), batch3,
                Wf1, bf1.reshape(1, G), Wf2p, bf2p)
    return outp[:, :2]


# speedup vs baseline: 1.0477x; 1.0477x over previous
import functools

import jax
import jax.numpy as jnp
from jax import lax
from jax.experimental import pallas as pl
from jax.experimental.pallas import tpu as pltpu
from jax.experimental.pallas import tpu_sc as plsc

N = 10000
E = 320000
D = 128
G = 64
NC, NS, L = 2, 16, 16

CH = 128
NCHUNK = 79
E_PAD = 32 * NCHUNK * CH
SCHUNK = 2 * NCHUNK
K0 = 120
K1 = SCHUNK - K0
KMAX = max(K0, K1)
SLAB = K0 + KMAX
NROWS = 10112
ZROWS = 632
BN = 1000
NBLK = N // BN


@functools.cache
def _sc_scatter_kernel():
    mesh = plsc.VectorSubcoreMesh(core_axis_name="c", subcore_axis_name="s")
    return functools.partial(
        pl.kernel,
        out_type=jax.ShapeDtypeStruct((NC, NROWS, D), jnp.float32),
        mesh=mesh,
        scratch_types=[
            pltpu.VMEM_SHARED((NROWS, D), jnp.float32),
            pltpu.VMEM((KMAX, CH), jnp.int32),
            pltpu.VMEM((KMAX, CH), jnp.int32),
            pltpu.VMEM((CH, D), jnp.float32),
            pltpu.SemaphoreType.DMA,
        ],
    )(_sc_scatter_body)


def _sc_scatter_body(table, srcC, dstC, zer, acc3,
                     acc_sp, srcv, dstv, gbuf, sem):
    c = lax.axis_index("c")
    s = lax.axis_index("s")
    start = jnp.where(c == 0, 0, K0)
    count = jnp.where(c == 0, K0, K1)
    pltpu.sync_copy(srcC.at[s, pl.ds(start, KMAX)], srcv)
    pltpu.sync_copy(dstC.at[s, pl.ds(start, KMAX)], dstv)
    pltpu.sync_copy(zer, acc_sp.at[pl.ds(s * ZROWS, ZROWS)])
    plsc.subcore_barrier()

    def body(j, _):
        pltpu.async_copy(table.at[srcv.at[j]], gbuf, sem).wait()
        pltpu.sync_copy(gbuf, acc_sp.at[dstv.at[j]], add=True)
        return 0

    lax.fori_loop(0, count, body, 0)
    plsc.subcore_barrier()
    pltpu.sync_copy(acc_sp.at[pl.ds(s * ZROWS, ZROWS)],
                    acc3.at[c, pl.ds(s * ZROWS, ZROWS)])


EPT = E_PAD // 32
DROWS = NROWS // NS


@functools.cache
def _sc_degree_kernel():
    mesh = plsc.VectorSubcoreMesh(core_axis_name="c", subcore_axis_name="s")
    return functools.partial(
        pl.kernel,
        out_type=jax.ShapeDtypeStruct((NC, NROWS, L), jnp.float32),
        mesh=mesh,
        compiler_params=pltpu.CompilerParams(needs_layout_passes=False),
        scratch_types=[
            pltpu.VMEM_SHARED((NS * NROWS,), jnp.float32),
            pltpu.VMEM((NROWS,), jnp.float32),
            pltpu.VMEM((DROWS + 8,), jnp.float32),
            pltpu.VMEM((DROWS + 8,), jnp.float32),
            pltpu.VMEM((EPT,), jnp.int32),
            pltpu.VMEM((DROWS, L), jnp.float32),
        ],
    )(_sc_degree_body)


def _sc_degree_body(dstF, deg3, tabs_sp, hist_v, sum_v, tmp_v, dstv, rep_v):
    c = lax.axis_index("c")
    s = lax.axis_index("s")
    wid = s * NC + c
    pltpu.sync_copy(dstF.at[wid], dstv)
    zeros16 = jnp.zeros((L,), jnp.float32)
    ones16 = jnp.ones((L,), jnp.float32)

    def zbody(j, _):
        hist_v[pl.ds(j * L, L)] = zeros16
        return 0

    lax.fori_loop(0, NROWS // L, zbody, 0)

    def hbody(j, _):
        idx = dstv[pl.ds(j * L, L)]
        plsc.addupdate_scatter(hist_v, [idx], ones16)
        return 0

    lax.fori_loop(0, EPT // L, hbody, 0)

    pltpu.sync_copy(hist_v, tabs_sp.at[pl.ds(s * NROWS, NROWS)])
    plsc.subcore_barrier()
    pltpu.sync_copy(tabs_sp.at[pl.ds(s * DROWS, DROWS)],
                    sum_v.at[pl.ds(0, DROWS)])
    for t in range(1, NS):
        pltpu.sync_copy(tabs_sp.at[pl.ds(t * NROWS + s * DROWS, DROWS)],
                        tmp_v.at[pl.ds(0, DROWS)])

        def abody(k, _):
            sl = pl.ds(k * L, L)
            sum_v[sl] = sum_v[sl] + tmp_v[sl]
            return 0

        lax.fori_loop(0, (DROWS + 8) // L, abody, 0)

    def rbody(j, _):
        v = plsc.load_gather(sum_v, [jnp.full((L,), j, jnp.int32)])
        plsc.store_scatter(
            rep_v, [jnp.full((L,), j, jnp.int32), lax.iota(jnp.int32, L)], v)
        return 0

    lax.fori_loop(0, DROWS, rbody, 0)
    pltpu.sync_copy(rep_v, deg3.at[c, pl.ds(s * DROWS, DROWS)])


def _dinv_block(degA, degB):
    deg = degA[0][:, :1] + degB[0][:, :1] + 1.0
    return lax.rsqrt(deg)


def _deg_specs():
    return [
        pl.BlockSpec((1, BN, L), lambda i: (0, i, 0)),
        pl.BlockSpec((1, BN, L), lambda i: (1, i, 0)),
    ]


def _tc1_body(x_ref, w1_ref, degA_ref, degB_ref, hs_ref):
    dinv = _dinv_block(degA_ref[...], degB_ref[...])
    h = jnp.dot(x_ref[...], w1_ref[...], preferred_element_type=jnp.float32)
    hs_ref[...] = h * dinv


def _tc1(x, W1, deg3):
    return pl.pallas_call(
        _tc1_body,
        grid=(NBLK,),
        in_specs=[
            pl.BlockSpec((BN, D), lambda i: (i, 0)),
            pl.BlockSpec((D, D), lambda i: (0, 0)),
        ] + _deg_specs(),
        out_specs=pl.BlockSpec((BN, D), lambda i: (i, 0)),
        out_shape=jax.ShapeDtypeStruct((N, D), jnp.float32),
    )(x, W1, deg3, deg3)


def _tc2_body(accA_ref, accB_ref, hs_ref, degA_ref, degB_ref,
              b1_ref, w2_ref, o_ref):
    dinv = _dinv_block(degA_ref[...], degB_ref[...])
    acc = accA_ref[0] + accB_ref[0]
    h1 = jax.nn.relu(dinv * (acc + hs_ref[...]) + b1_ref[...])
    o_ref[...] = jnp.dot(h1, w2_ref[...],
                         preferred_element_type=jnp.float32) * dinv


def _tc2(acc3, hs, deg3, b1, W2):
    return pl.pallas_call(
        _tc2_body,
        grid=(NBLK,),
        in_specs=[
            pl.BlockSpec((1, BN, D), lambda i: (0, i, 0)),
            pl.BlockSpec((1, BN, D), lambda i: (1, i, 0)),
            pl.BlockSpec((BN, D), lambda i: (i, 0)),
        ] + _deg_specs() + [
            pl.BlockSpec((1, D), lambda i: (0, 0)),
            pl.BlockSpec((D, D), lambda i: (0, 0)),
        ],
        out_specs=pl.BlockSpec((BN, D), lambda i: (i, 0)),
        out_shape=jax.ShapeDtypeStruct((N, D), jnp.float32),
    )(acc3, acc3, hs, deg3, deg3, b1, W2)


def _tc3_body(accA_ref, accB_ref, hs_ref, degA_ref, degB_ref,
              b2_ref, batch_ref, wf1_ref, bf1_ref, wf2_ref, bf2_ref,
              out_ref, pool_s, cnt_s):
    i = pl.program_id(0)
    dinv = _dinv_block(degA_ref[...], degB_ref[...])
    acc = accA_ref[0] + accB_ref[0]
    h2 = jax.nn.relu(dinv * (acc + hs_ref[...]) + b2_ref[...])

    @pl.when(i == 0)
    def _():
        pool_s[...] = jnp.zeros_like(pool_s)
        cnt_s[...] = jnp.zeros_like(cnt_s)

    ids = batch_ref[0, 0, :]
    gids = lax.broadcasted_iota(jnp.int32, (G, BN), 0)
    onehot = (ids[None, :] == gids).astype(jnp.float32)
    pool_s[...] += jnp.dot(onehot, h2, preferred_element_type=jnp.float32)
    cnt_s[...] += jnp.dot(onehot, jnp.ones((BN, D), jnp.float32),
                          preferred_element_type=jnp.float32)

    @pl.when(i == NBLK - 1)
    def _():
        pooled = pool_s[...] / jnp.maximum(cnt_s[...], 1.0)
        o = jax.nn.relu(
            jnp.dot(pooled, wf1_ref[...], preferred_element_type=jnp.float32)
            + bf1_ref[...])
        out_ref[...] = (
            jnp.dot(o, wf2_ref[...], preferred_element_type=jnp.float32)
            + bf2_ref[...])


def _tc3(acc3, hs, deg3, b2, batch3, Wf1, bf1, Wf2p, bf2p):
    return pl.pallas_call(
        _tc3_body,
        grid=(NBLK,),
        in_specs=[
            pl.BlockSpec((1, BN, D), lambda i: (0, i, 0)),
            pl.BlockSpec((1, BN, D), lambda i: (1, i, 0)),
            pl.BlockSpec((BN, D), lambda i: (i, 0)),
        ] + _deg_specs() + [
            pl.BlockSpec((1, D), lambda i: (0, 0)),
            pl.BlockSpec((1, 1, BN), lambda i: (i, 0, 0)),
            pl.BlockSpec((D, G), lambda i: (0, 0)),
            pl.BlockSpec((1, G), lambda i: (0, 0)),
            pl.BlockSpec((G, D), lambda i: (0, 0)),
            pl.BlockSpec((1, D), lambda i: (0, 0)),
        ],
        out_specs=pl.BlockSpec((G, D), lambda i: (0, 0)),
        out_shape=jax.ShapeDtypeStruct((G, D), jnp.float32),
        scratch_shapes=[
            pltpu.VMEM((G, D), jnp.float32),
            pltpu.VMEM((G, D), jnp.float32),
        ],
    )(acc3, acc3, hs, deg3, deg3, b2, batch3, Wf1, bf1, Wf2p, bf2p)


def kernel(x, edge_index, batch, W1, b1, W2, b2, Wf1, bf1, Wf2, bf2):
    src = edge_index[0].astype(jnp.int32)
    dst = edge_index[1].astype(jnp.int32)
    npad = E_PAD - E
    src_pad = jnp.concatenate([src, jnp.zeros((npad,), jnp.int32)])
    dst_pad = jnp.concatenate([dst, jnp.full((npad,), N, jnp.int32)])

    src2 = src_pad.reshape(NS, SCHUNK, CH)
    dst2 = dst_pad.reshape(NS, SCHUNK, CH)
    padc = jnp.zeros((NS, SLAB - SCHUNK, CH), jnp.int32)
    srcC = jnp.concatenate([src2, padc], axis=1)
    dstC = jnp.concatenate([dst2, jnp.full((NS, SLAB - SCHUNK, CH), N,
                                           jnp.int32)], axis=1)

    dstF = dst_pad.reshape(32, EPT)
    zer128 = jnp.zeros((ZROWS, D), jnp.float32)
    batch3 = batch.astype(jnp.int32).reshape(NBLK, 1, BN)

    scat = _sc_scatter_kernel()
    deg3 = _sc_degree_kernel()(dstF)
    hs = _tc1(x, W1, deg3)
    acc3 = scat(hs, srcC, dstC, zer128)
    hs2 = _tc2(acc3, hs, deg3, b1.reshape(1, D), W2)
    acc23 = scat(hs2, srcC, dstC, zer128)
    Wf2p = jnp.zeros((G, D), jnp.float32).at[:, :2].set(Wf2)
    bf2p = jnp.zeros((1, D), jnp.float32).at[:, :2].set(bf2.reshape(1, 2))
    outp = _tc3(acc23, hs2, deg3, b2.reshape(1, D), batch3,
                Wf1, bf1.reshape(1, G), Wf2p, bf2p)
    return outp[:, :2]
